# Initial kernel scaffold; baseline (speedup 1.0000x reference)
import jax
import jax.numpy as jnp
from jax.experimental import pallas as pl


def kernel(x, edge_index, W1, b1, W2, b2):
    raise NotImplementedError("write your pallas kernel here")


# speedup vs baseline: 5.1275x; 5.1275x over previous
import functools

import jax
import jax.numpy as jnp
from jax import lax
from jax.experimental import pallas as pl
from jax.experimental.pallas import tpu as pltpu
from jax.experimental.pallas import tpu_sc as plsc

N = 10000
D = 128
D2 = D // 2
NS = 16
L = 16
K = 128
E = 320000
T16 = 160
EP = NS * T16 * K
CB = T16 + 1
CLEN = CB * K
HALF = N // 2
AGG_ROWS = 5120
ZR = AGG_ROWS // NS

_mesh = plsc.VectorSubcoreMesh(core_axis_name="c", subcore_axis_name="s")
_sc_params = pltpu.CompilerParams(needs_layout_passes=False,
                                 use_tc_tiling_on_sc=False)


def _compact_body(srcr, dstr, csrc_hbm, cdst_hbm, nch_hbm,
                  src_t, dst_t, csrc, cdst, nchbuf):
    c = lax.axis_index("c")
    s = lax.axis_index("s")
    lo = c * HALF

    pltpu.sync_copy(srcr.at[pl.ds(s * T16, T16)], src_t)
    pltpu.sync_copy(dstr.at[pl.ds(s * T16, T16)], dst_t)

    iota = jnp.arange(L, dtype=jnp.int32)

    def chunk(r, off):
        for j in range(K // L):
            dv = dst_t[r, pl.ds(j * L, L)]
            m = (dv >= lo) & (dv < lo + HALF)
            scan = plsc.cumsum(m.astype(jnp.int32))
            pos = jnp.where(m, off + scan - 1, CLEN + iota)
            plsc.store_scatter(cdst, [pos], dv - lo)
            sv = src_t[r, pl.ds(j * L, L)]
            plsc.store_scatter(csrc, [pos], sv)
            off = off + scan[L - 1]
        return off

    cnt = lax.fori_loop(0, T16, chunk, jnp.int32(0))

    for k in range(K // L):
        csrc[pl.ds(cnt + k * L, L)] = iota
        cdst[pl.ds(cnt + k * L, L)] = HALF + iota

    nch = (cnt + K - 1) // K
    nchbuf[...] = jnp.zeros((L,), jnp.int32) + nch
    pltpu.sync_copy(nchbuf, nch_hbm.at[c, s])
    pltpu.sync_copy(csrc.at[pl.ds(0, CLEN)], csrc_hbm.at[c, s])
    pltpu.sync_copy(cdst.at[pl.ds(0, CLEN)], cdst_hbm.at[c, s])


_sc_compact = pl.kernel(
    _compact_body,
    out_type=(jax.ShapeDtypeStruct((2, NS, CLEN), jnp.int32),
              jax.ShapeDtypeStruct((2, NS, CLEN), jnp.int32),
              jax.ShapeDtypeStruct((2, NS, L), jnp.int32)),
    mesh=_mesh,
    compiler_params=_sc_params,
    scratch_types=[
        pltpu.VMEM((T16, K), jnp.int32),
        pltpu.VMEM((T16, K), jnp.int32),
        pltpu.VMEM((CLEN + L,), jnp.int32),
        pltpu.VMEM((CLEN + L,), jnp.int32),
        pltpu.VMEM((L,), jnp.int32),
    ],
)


def _make_agg(p, with_counts):
    def body(h_hbm, csrc_hbm, cdst_hbm, nch_hbm, *refs):
        if with_counts:
            (out_hbm, cnt_hbm, csrc_l, cdst_l, rows, idxg, ones, zcnt, idx,
             nch_s, agg_sh, cnt_sh, sem) = refs
        else:
            (out_hbm, csrc_l, cdst_l, rows, idxg, idx,
             nch_s, agg_sh, sem) = refs
        c = lax.axis_index("c")
        s = lax.axis_index("s")

        iota = jnp.arange(L, dtype=jnp.int32)

        def fill_idx(base, n):
            for j in range(K // L):
                off = 16 * j if 16 * j < n else 16 * j - n
                idx[pl.ds(16 * j, L)] = base + off + iota

        @pl.loop(0, K)
        def _(i):
            for j in range(D2 // L):
                rows[i, pl.ds(j * L, L)] = jnp.zeros((L,), jnp.float32)

        if with_counts:
            @pl.loop(0, K)
            def _(i):
                zcnt[i, :] = jnp.zeros((L,), jnp.float32)

            @pl.loop(0, K)
            def _(i):
                ones[i, :] = jnp.ones((L,), jnp.float32)

        zb = s * ZR
        for base, n in ((0, K), (K, K), (2 * K, ZR - 2 * K)):
            fill_idx(zb + base, n)
            pltpu.sync_copy(rows, agg_sh.at[idx])
            if with_counts:
                pltpu.sync_copy(zcnt, cnt_sh.at[idx])

        pltpu.sync_copy(csrc_hbm.at[c, s], csrc_l)
        pltpu.sync_copy(cdst_hbm.at[c, s], cdst_l)
        pltpu.async_copy(nch_hbm.at[c, s], nch_s, sem).wait()
        nch = nch_s[...][0]
        plsc.subcore_barrier()

        def _step(t, carry):
            for j in range(K // L):
                v = csrc_l[t, pl.ds(j * L, L)]
                idxg[pl.ds(j * L, L)] = v * 2 + p
            pltpu.async_copy(h_hbm.at[idxg], rows, sem).wait()
            pltpu.sync_copy(rows, agg_sh.at[cdst_l.at[t]], add=True)
            if with_counts:
                pltpu.sync_copy(ones, cnt_sh.at[cdst_l.at[t]], add=True)
            return carry

        lax.fori_loop(0, nch, _step, jnp.int32(0))
        plsc.subcore_barrier()

        hb = c * AGG_ROWS + zb
        for base, n in ((0, K), (K, K), (2 * K, ZR - 2 * K)):
            fill_idx(zb + base, n)
            pltpu.async_copy(agg_sh.at[idx], rows, sem).wait()
            pltpu.sync_copy(rows.at[:n], out_hbm.at[pl.ds(hb + base, n)])
            if with_counts:
                pltpu.async_copy(cnt_sh.at[idx], zcnt, sem).wait()
                pltpu.sync_copy(zcnt.at[:n], cnt_hbm.at[pl.ds(hb + base, n)])

    agg_ty = jax.ShapeDtypeStruct((2 * AGG_ROWS, D2), jnp.float32)
    scratch = [
        pltpu.VMEM((CB, K), jnp.int32),
        pltpu.VMEM((CB, K), jnp.int32),
        pltpu.VMEM((K, D2), jnp.float32),
        pltpu.VMEM((K,), jnp.int32),
    ]
    if with_counts:
        out_type = (agg_ty,
                    jax.ShapeDtypeStruct((2 * AGG_ROWS, L), jnp.float32))
        scratch += [
            pltpu.VMEM((K, L), jnp.float32),
            pltpu.VMEM((K, L), jnp.float32),
        ]
    else:
        out_type = agg_ty
    scratch += [
        pltpu.VMEM((K,), jnp.int32),
        pltpu.VMEM((L,), jnp.int32),
        pltpu.VMEM_SHARED((AGG_ROWS, D2), jnp.float32),
    ]
    if with_counts:
        scratch += [pltpu.VMEM_SHARED((AGG_ROWS, L), jnp.float32)]
    scratch += [pltpu.SemaphoreType.DMA]

    return pl.kernel(body, out_type=out_type, mesh=_mesh,
                     compiler_params=_sc_params, scratch_types=scratch)


_sc_agg_p0 = _make_agg(0, True)
_sc_agg_p1 = _make_agg(1, False)
_sc_agg_p0nc = _make_agg(0, False)

_R = 1000


def _combine_body(do_relu, p, h, c0, w, b, o):
    cnt = c0[:, 0:1]
    hn = (p[...] + h[...]) / (cnt + 1.0)
    acc = jnp.dot(hn, w[...], preferred_element_type=jnp.float32) + b[...]
    o[...] = jnp.maximum(acc, 0.0) if do_relu else acc


def _combine(p, h, c0, w, b, do_relu):
    return pl.pallas_call(
        functools.partial(_combine_body, do_relu),
        grid=(N // _R,),
        in_specs=[
            pl.BlockSpec((_R, D), lambda i: (i, 0)),
            pl.BlockSpec((_R, D), lambda i: (i, 0)),
            pl.BlockSpec((_R, L), lambda i: (i, 0)),
            pl.BlockSpec((D, D), lambda i: (0, 0)),
            pl.BlockSpec((1, D), lambda i: (0, 0)),
        ],
        out_specs=pl.BlockSpec((_R, D), lambda i: (i, 0)),
        out_shape=jax.ShapeDtypeStruct((N, D), jnp.float32),
    )(p, h, c0, w, b)


def _unpad(a):
    return jnp.concatenate([a[:HALF], a[AGG_ROWS:AGG_ROWS + HALF]])


def kernel(x, edge_index, W1, b1, W2, b2):
    src = edge_index[0].astype(jnp.int32)
    dst = edge_index[1].astype(jnp.int32)
    pad = EP - E
    src_p = jnp.concatenate([src, jnp.zeros((pad,), jnp.int32)])
    dst_p = jnp.concatenate([dst, jnp.full((pad,), N, jnp.int32)])
    src_r = src_p.reshape(NS * T16, K)
    dst_r = dst_p.reshape(NS * T16, K)

    csrc, cdst, nch = _sc_compact(src_r, dst_r)
    csrc = csrc.reshape(2, NS, CB, K)
    cdst = cdst.reshape(2, NS, CB, K)

    xv = x.reshape(2 * N, D2)
    a0, cnt1p = _sc_agg_p0(xv, csrc, cdst, nch)
    a1 = _sc_agg_p1(xv, csrc, cdst, nch)
    agg1 = jnp.concatenate([_unpad(a0), _unpad(a1)], axis=1)
    cnt1 = _unpad(cnt1p)
    h1 = _combine(agg1, x, cnt1, W1, b1.reshape(1, D), True)

    hv = h1.reshape(2 * N, D2)
    b0 = _sc_agg_p0nc(hv, csrc, cdst, nch)
    b1h = _sc_agg_p1(hv, csrc, cdst, nch)
    agg2 = jnp.concatenate([_unpad(b0), _unpad(b1h)], axis=1)
    out = _combine(agg2, h1, cnt1, W2, b2.reshape(1, D), False)
    return out
